# Initial kernel scaffold; baseline (speedup 1.0000x reference)
import jax
import jax.numpy as jnp
from jax.experimental import pallas as pl


def kernel(node_attr, edge_index, edge_attr, global_attr, W1, b1, W2, b2):
    raise NotImplementedError("write your pallas kernel here")


# speedup vs baseline: 4.6531x; 4.6531x over previous
import functools

import jax
import jax.numpy as jnp
from jax import lax
from jax.experimental import pallas as pl
from jax.experimental.pallas import tpu as pltpu
from jax.experimental.pallas import tpu_sc as plsc

F32 = jnp.float32

_NC = 2
_NS = 16
_NW = _NC * _NS

_CH = 1000
_SLICES = tuple((o, 128) for o in range(0, 896, 128)) + ((896, 104),)


def _prep_body(node_ref, w1s_ref, w1r_ref, g_ref, w1g_ref, b1_ref,
               s_ref, r_ref, c4_ref):
    n = node_ref[...]
    s_ref[...] = jnp.dot(n, w1s_ref[...], preferred_element_type=F32)
    r_ref[...] = jnp.dot(n, w1r_ref[...], preferred_element_type=F32)
    c = jnp.dot(g_ref[...], w1g_ref[...], preferred_element_type=F32) + b1_ref[...]
    c4_ref[...] = jnp.concatenate([c, c, c, c], axis=1)


def _mlp_body(e4_ref, gs4_ref, gr4_ref, w1e_ref, c4_ref, w2_ref, b24_ref,
              out_ref):
    x = jnp.dot(e4_ref[...], w1e_ref[...], preferred_element_type=F32)
    x = x + gs4_ref[...] + gr4_ref[...] + c4_ref[...]
    h = jnp.maximum(x, 0.0)
    out_ref[...] = jnp.dot(h, w2_ref[...], preferred_element_type=F32) + b24_ref[...]


def _make_gather(num_edges, latent):
    per_w = num_edges // _NW
    nchunk = per_w // _CH
    mesh = plsc.VectorSubcoreMesh(core_axis_name="c", subcore_axis_name="s")

    @functools.partial(
        pl.kernel,
        mesh=mesh,
        out_type=[
            jax.ShapeDtypeStruct((num_edges, latent), F32),
            jax.ShapeDtypeStruct((num_edges, latent), F32),
        ],
        scratch_types=[
            pltpu.VMEM((_CH,), jnp.int32),
            pltpu.VMEM((_CH,), jnp.int32),
            pltpu.VMEM((_CH, latent), F32),
            pltpu.VMEM((_CH, latent), F32),
            pltpu.SemaphoreType.DMA,
            pltpu.SemaphoreType.DMA,
        ],
        compiler_params=pltpu.CompilerParams(use_tc_tiling_on_sc=False),
    )
    def gather_call(s_hbm, r_hbm, src_hbm, dst_hbm, out_s, out_r,
                    idx_s, idx_d, buf_s, buf_r, sem_s, sem_r):
        wid = lax.axis_index("s") * _NC + lax.axis_index("c")
        for k in range(nchunk):
            base = wid * per_w + k * _CH
            pltpu.sync_copy(src_hbm.at[pl.ds(base, _CH)], idx_s)
            pltpu.sync_copy(dst_hbm.at[pl.ds(base, _CH)], idx_d)
            handles = []
            for off, sz in _SLICES:
                handles.append(pltpu.async_copy(
                    s_hbm.at[idx_s.at[pl.ds(off, sz)]],
                    buf_s.at[pl.ds(off, sz)], sem_s))
                handles.append(pltpu.async_copy(
                    r_hbm.at[idx_d.at[pl.ds(off, sz)]],
                    buf_r.at[pl.ds(off, sz)], sem_r))
            for h in handles:
                h.wait()
            pltpu.sync_copy(buf_s, out_s.at[pl.ds(base, _CH)])
            pltpu.sync_copy(buf_r, out_r.at[pl.ds(base, _CH)])

    return gather_call


def kernel(node_attr, edge_index, edge_attr, global_attr, W1, b1, W2, b2):
    n_nodes, d_feat = node_attr.shape
    num_edges, d_edge = edge_attr.shape
    latent = W1.shape[1]
    d_out = W2.shape[1]

    src = edge_index[0].astype(jnp.int32)
    dst = edge_index[1].astype(jnp.int32)
    W1e = W1[:d_edge]
    W1s = W1[d_edge:d_edge + d_feat]
    W1r = W1[d_edge + d_feat:d_edge + 2 * d_feat]
    W1g = W1[d_edge + 2 * d_feat:]

    S, R, c4 = pl.pallas_call(
        _prep_body,
        out_shape=[
            jax.ShapeDtypeStruct((n_nodes, latent), F32),
            jax.ShapeDtypeStruct((n_nodes, latent), F32),
            jax.ShapeDtypeStruct((1, 4 * latent), F32),
        ],
    )(node_attr, W1s, W1r, global_attr, W1g, b1.reshape(1, latent))

    GS, GR = _make_gather(num_edges, latent)(S, R, src, dst)

    eye4 = jnp.eye(4, dtype=F32)
    W1e_bd = jnp.kron(eye4, W1e)
    W2_bd = jnp.kron(eye4, W2)
    b24 = jnp.tile(b2, 4).reshape(1, 4 * d_out)

    rows = num_edges // 4
    block = 4000
    grid = rows // block
    out4 = pl.pallas_call(
        _mlp_body,
        grid=(grid,),
        in_specs=[
            pl.BlockSpec((block, 4 * d_edge), lambda i: (i, 0)),
            pl.BlockSpec((block, 4 * latent), lambda i: (i, 0)),
            pl.BlockSpec((block, 4 * latent), lambda i: (i, 0)),
            pl.BlockSpec((4 * d_edge, 4 * latent), lambda i: (0, 0)),
            pl.BlockSpec((1, 4 * latent), lambda i: (0, 0)),
            pl.BlockSpec((4 * latent, 4 * d_out), lambda i: (0, 0)),
            pl.BlockSpec((1, 4 * d_out), lambda i: (0, 0)),
        ],
        out_specs=pl.BlockSpec((block, 4 * d_out), lambda i: (i, 0)),
        out_shape=jax.ShapeDtypeStruct((rows, 4 * d_out), F32),
    )(edge_attr.reshape(rows, 4 * d_edge),
      GS.reshape(rows, 4 * latent),
      GR.reshape(rows, 4 * latent),
      W1e_bd, c4, W2_bd, b24)

    return out4.reshape(num_edges, d_out)


# speedup vs baseline: 4.8630x; 1.0451x over previous
import functools

import jax
import jax.numpy as jnp
from jax import lax
from jax.experimental import pallas as pl
from jax.experimental.pallas import tpu as pltpu
from jax.experimental.pallas import tpu_sc as plsc

F32 = jnp.float32

_NC = 2
_NS = 16
_NW = _NC * _NS

_CH = 400
_SLICES = ((0, 128), (128, 128), (256, 128), (384, 16))


def _prep_body(node_ref, w1s_ref, w1r_ref, g_ref, w1g_ref, b1_ref,
               s_ref, r_ref, c4_ref):
    n = node_ref[...]
    s_ref[...] = jnp.dot(n, w1s_ref[...], preferred_element_type=F32)
    r_ref[...] = jnp.dot(n, w1r_ref[...], preferred_element_type=F32)
    c = jnp.dot(g_ref[...], w1g_ref[...], preferred_element_type=F32) + b1_ref[...]
    c4_ref[...] = jnp.concatenate([c, c, c, c], axis=1)


def _mlp_body(e4_ref, g_ref, w1e_ref, c4_ref, w2_ref, b24_ref, out_ref):
    x = jnp.dot(e4_ref[...], w1e_ref[...], preferred_element_type=F32)
    x = x + g_ref[...] + c4_ref[...]
    h = jnp.maximum(x, 0.0)
    out_ref[...] = jnp.dot(h, w2_ref[...], preferred_element_type=F32) + b24_ref[...]


def _make_gather(num_edges, latent):
    per_w = num_edges // _NW
    nchunk = per_w // _CH
    rows_ch = _CH // 4
    mesh = plsc.VectorSubcoreMesh(core_axis_name="c", subcore_axis_name="s")

    @functools.partial(
        pl.kernel,
        mesh=mesh,
        out_type=jax.ShapeDtypeStruct((num_edges // 4, 4 * latent), F32),
        scratch_types=[
            pltpu.VMEM((per_w,), jnp.int32),
            pltpu.VMEM((per_w,), jnp.int32),
            pltpu.VMEM((_CH, latent), F32),
            pltpu.VMEM((_CH, latent), F32),
            pltpu.VMEM((_CH, latent), F32),
            pltpu.VMEM((_CH, latent), F32),
            pltpu.VMEM((rows_ch, 4 * latent), F32),
            pltpu.VMEM((rows_ch, 4 * latent), F32),
            pltpu.SemaphoreType.DMA,
            pltpu.SemaphoreType.DMA,
            pltpu.SemaphoreType.DMA,
            pltpu.SemaphoreType.DMA,
        ],
        compiler_params=pltpu.CompilerParams(use_tc_tiling_on_sc=False),
    )
    def gather_call(s_hbm, r_hbm, src_hbm, dst_hbm, out_g,
                    idx_s, idx_d, buf_s0, buf_r0, buf_s1, buf_r1,
                    pk0, pk1, sem0, sem1, sem_w0, sem_w1):
        wid = lax.axis_index("s") * _NC + lax.axis_index("c")
        ebase = wid * per_w
        pltpu.sync_copy(src_hbm.at[pl.ds(ebase, per_w)], idx_s)
        pltpu.sync_copy(dst_hbm.at[pl.ds(ebase, per_w)], idx_d)

        bufs = ((buf_s0, buf_r0, pk0, sem0, sem_w0),
                (buf_s1, buf_r1, pk1, sem1, sem_w1))
        pending = {}
        wpending = {}

        def fire(k):
            buf_s, buf_r, _, sem, _ = bufs[k % 2]
            hs = []
            for off, sz in _SLICES:
                lo = k * _CH + off
                hs.append(pltpu.async_copy(
                    s_hbm.at[idx_s.at[pl.ds(lo, sz)]],
                    buf_s.at[pl.ds(off, sz)], sem))
                hs.append(pltpu.async_copy(
                    r_hbm.at[idx_d.at[pl.ds(lo, sz)]],
                    buf_r.at[pl.ds(off, sz)], sem))
            pending[k % 2] = hs

        def drain_pack_write(k):
            buf_s, buf_r, pk, _, sem_w = bufs[k % 2]
            for h in pending.pop(k % 2):
                h.wait()
            if k % 2 in wpending:
                wpending.pop(k % 2).wait()

            def body(r, carry):
                for q in range(4):
                    for hh in range(2):
                        a = buf_s[4 * r + q, pl.ds(16 * hh, 16)]
                        b = buf_r[4 * r + q, pl.ds(16 * hh, 16)]
                        pk[r, pl.ds(32 * q + 16 * hh, 16)] = a + b
                return carry

            lax.fori_loop(0, rows_ch, body, 0)
            rbase = (ebase + k * _CH) // 4
            wpending[k % 2] = pltpu.async_copy(
                pk, out_g.at[pl.ds(rbase, rows_ch)], sem_w)

        fire(0)
        for k in range(1, nchunk):
            fire(k)
            drain_pack_write(k - 1)
        drain_pack_write(nchunk - 1)
        for h in wpending.values():
            h.wait()

    return gather_call


def kernel(node_attr, edge_index, edge_attr, global_attr, W1, b1, W2, b2):
    n_nodes, d_feat = node_attr.shape
    num_edges, d_edge = edge_attr.shape
    latent = W1.shape[1]
    d_out = W2.shape[1]

    src = edge_index[0].astype(jnp.int32)
    dst = edge_index[1].astype(jnp.int32)
    W1e = W1[:d_edge]
    W1s = W1[d_edge:d_edge + d_feat]
    W1r = W1[d_edge + d_feat:d_edge + 2 * d_feat]
    W1g = W1[d_edge + 2 * d_feat:]

    S, R, c4 = pl.pallas_call(
        _prep_body,
        out_shape=[
            jax.ShapeDtypeStruct((n_nodes, latent), F32),
            jax.ShapeDtypeStruct((n_nodes, latent), F32),
            jax.ShapeDtypeStruct((1, 4 * latent), F32),
        ],
    )(node_attr, W1s, W1r, global_attr, W1g, b1.reshape(1, latent))

    G = _make_gather(num_edges, latent)(S, R, src, dst)

    eye4 = jnp.eye(4, dtype=F32)
    W1e_bd = jnp.kron(eye4, W1e)
    W2_bd = jnp.kron(eye4, W2)
    b24 = jnp.tile(b2, 4).reshape(1, 4 * d_out)

    rows = num_edges // 4
    block = 4000
    grid = rows // block
    out4 = pl.pallas_call(
        _mlp_body,
        grid=(grid,),
        in_specs=[
            pl.BlockSpec((block, 4 * d_edge), lambda i: (i, 0)),
            pl.BlockSpec((block, 4 * latent), lambda i: (i, 0)),
            pl.BlockSpec((4 * d_edge, 4 * latent), lambda i: (0, 0)),
            pl.BlockSpec((1, 4 * latent), lambda i: (0, 0)),
            pl.BlockSpec((4 * latent, 4 * d_out), lambda i: (0, 0)),
            pl.BlockSpec((1, 4 * d_out), lambda i: (0, 0)),
        ],
        out_specs=pl.BlockSpec((block, 4 * d_out), lambda i: (i, 0)),
        out_shape=jax.ShapeDtypeStruct((rows, 4 * d_out), F32),
    )(edge_attr.reshape(rows, 4 * d_edge), G, W1e_bd, c4, W2_bd, b24)

    return out4.reshape(num_edges, d_out)


# speedup vs baseline: 4.8692x; 1.0013x over previous
import functools

import jax
import jax.numpy as jnp
from jax import lax
from jax.experimental import pallas as pl
from jax.experimental.pallas import tpu as pltpu
from jax.experimental.pallas import tpu_sc as plsc

F32 = jnp.float32

_NC = 2
_NS = 16
_NW = _NC * _NS

_CH = 400
_SLICES = ((0, 128), (128, 128), (256, 128), (384, 16))


def _prep_body(node_ref, w1s_ref, w1r_ref, g_ref, w1g_ref, b1_ref,
               s_ref, r_ref, c4_ref):
    n = node_ref[...]
    s_ref[...] = jnp.dot(n, w1s_ref[...], preferred_element_type=F32)
    r_ref[...] = jnp.dot(n, w1r_ref[...], preferred_element_type=F32)
    c = jnp.dot(g_ref[...], w1g_ref[...], preferred_element_type=F32) + b1_ref[...]
    c4_ref[...] = jnp.concatenate([c, c, c, c], axis=1)


def _mlp_body(e4_ref, g_ref, w1e_ref, c4_ref, w2_ref, b24_ref, out_ref):
    x = jnp.dot(e4_ref[...], w1e_ref[...], preferred_element_type=F32)
    x = x + g_ref[...] + c4_ref[...]
    h = jnp.maximum(x, 0.0)
    out_ref[...] = jnp.dot(h, w2_ref[...], preferred_element_type=F32) + b24_ref[...]


def _make_gather(num_edges, latent):
    per_w = num_edges // _NW
    nchunk = per_w // _CH
    rows_ch = _CH // 4
    mesh = plsc.VectorSubcoreMesh(core_axis_name="c", subcore_axis_name="s")

    @functools.partial(
        pl.kernel,
        mesh=mesh,
        out_type=jax.ShapeDtypeStruct((num_edges * latent,), F32),
        scratch_types=[
            pltpu.VMEM((per_w,), jnp.int32),
            pltpu.VMEM((per_w,), jnp.int32),
            pltpu.VMEM((_CH, latent), F32),
            pltpu.VMEM((_CH, latent), F32),
            pltpu.VMEM((_CH, latent), F32),
            pltpu.VMEM((_CH, latent), F32),
            pltpu.VMEM((_CH * latent,), F32),
            pltpu.VMEM((_CH * latent,), F32),
            pltpu.SemaphoreType.DMA,
            pltpu.SemaphoreType.DMA,
            pltpu.SemaphoreType.DMA,
            pltpu.SemaphoreType.DMA,
        ],
        compiler_params=pltpu.CompilerParams(use_tc_tiling_on_sc=False),
    )
    def gather_call(s_hbm, r_hbm, src_hbm, dst_hbm, out_g,
                    idx_s, idx_d, buf_s0, buf_r0, buf_s1, buf_r1,
                    pk0, pk1, sem0, sem1, sem_w0, sem_w1):
        wid = lax.axis_index("s") * _NC + lax.axis_index("c")
        ebase = wid * per_w
        pltpu.sync_copy(src_hbm.at[pl.ds(ebase, per_w)], idx_s)
        pltpu.sync_copy(dst_hbm.at[pl.ds(ebase, per_w)], idx_d)

        bufs = ((buf_s0, buf_r0, pk0, sem0, sem_w0),
                (buf_s1, buf_r1, pk1, sem1, sem_w1))
        pending = {}
        wpending = {}

        def fire(k):
            buf_s, buf_r, _, sem, _ = bufs[k % 2]
            hs = []
            for off, sz in _SLICES:
                lo = k * _CH + off
                hs.append(pltpu.async_copy(
                    s_hbm.at[idx_s.at[pl.ds(lo, sz)]],
                    buf_s.at[pl.ds(off, sz)], sem))
                hs.append(pltpu.async_copy(
                    r_hbm.at[idx_d.at[pl.ds(lo, sz)]],
                    buf_r.at[pl.ds(off, sz)], sem))
            pending[k % 2] = hs

        def drain_pack_write(k):
            buf_s, buf_r, pk, _, sem_w = bufs[k % 2]
            for h in pending.pop(k % 2):
                h.wait()
            if k % 2 in wpending:
                wpending.pop(k % 2).wait()

            def body(r, carry):
                for q in range(4):
                    for hh in range(2):
                        a = buf_s[4 * r + q, pl.ds(16 * hh, 16)]
                        b = buf_r[4 * r + q, pl.ds(16 * hh, 16)]
                        pk[pl.ds(128 * r + 32 * q + 16 * hh, 16)] = a + b
                return carry

            lax.fori_loop(0, rows_ch, body, 0)
            fbase = (ebase + k * _CH) * latent
            wpending[k % 2] = pltpu.async_copy(
                pk, out_g.at[pl.ds(fbase, _CH * latent)], sem_w)

        fire(0)
        for k in range(1, nchunk):
            fire(k)
            drain_pack_write(k - 1)
        drain_pack_write(nchunk - 1)
        for h in wpending.values():
            h.wait()

    return gather_call


def kernel(node_attr, edge_index, edge_attr, global_attr, W1, b1, W2, b2):
    n_nodes, d_feat = node_attr.shape
    num_edges, d_edge = edge_attr.shape
    latent = W1.shape[1]
    d_out = W2.shape[1]

    src = edge_index[0].astype(jnp.int32)
    dst = edge_index[1].astype(jnp.int32)
    W1e = W1[:d_edge]
    W1s = W1[d_edge:d_edge + d_feat]
    W1r = W1[d_edge + d_feat:d_edge + 2 * d_feat]
    W1g = W1[d_edge + 2 * d_feat:]

    S, R, c4 = pl.pallas_call(
        _prep_body,
        out_shape=[
            jax.ShapeDtypeStruct((n_nodes, latent), F32),
            jax.ShapeDtypeStruct((n_nodes, latent), F32),
            jax.ShapeDtypeStruct((1, 4 * latent), F32),
        ],
    )(node_attr, W1s, W1r, global_attr, W1g, b1.reshape(1, latent))

    G = _make_gather(num_edges, latent)(S, R, src, dst)
    G = G.reshape(num_edges // 4, 4 * latent)

    eye4 = jnp.eye(4, dtype=F32)
    W1e_bd = jnp.kron(eye4, W1e)
    W2_bd = jnp.kron(eye4, W2)
    b24 = jnp.tile(b2, 4).reshape(1, 4 * d_out)

    rows = num_edges // 4
    block = 4000
    grid = rows // block
    out4 = pl.pallas_call(
        _mlp_body,
        grid=(grid,),
        in_specs=[
            pl.BlockSpec((block, 4 * d_edge), lambda i: (i, 0)),
            pl.BlockSpec((block, 4 * latent), lambda i: (i, 0)),
            pl.BlockSpec((4 * d_edge, 4 * latent), lambda i: (0, 0)),
            pl.BlockSpec((1, 4 * latent), lambda i: (0, 0)),
            pl.BlockSpec((4 * latent, 4 * d_out), lambda i: (0, 0)),
            pl.BlockSpec((1, 4 * d_out), lambda i: (0, 0)),
        ],
        out_specs=pl.BlockSpec((block, 4 * d_out), lambda i: (i, 0)),
        out_shape=jax.ShapeDtypeStruct((rows, 4 * d_out), F32),
    )(edge_attr.reshape(rows, 4 * d_edge), G, W1e_bd, c4, W2_bd, b24)

    return out4.reshape(num_edges, d_out)
